# baseline (speedup 1.0000x reference)
import functools

import jax
import jax.numpy as jnp
from jax import lax
from jax.experimental import pallas as pl
from jax.experimental.pallas import tpu as pltpu
from jax.experimental.pallas import tpu_sc as plsc

D = 32
B = 4096 * 50
NW = 32
B_PER_W = B // NW
CHUNK = 3200
NCHUNK = B_PER_W // CHUNK

_mesh = plsc.VectorSubcoreMesh(core_axis_name="c", subcore_axis_name="s")


@functools.partial(
    pl.kernel,
    mesh=_mesh,
    out_type=jax.ShapeDtypeStruct((B, D), jnp.float32),
    compiler_params=pltpu.CompilerParams(use_tc_tiling_on_sc=False),
    scratch_types=[
        pltpu.VMEM((CHUNK,), jnp.int32),
        pltpu.VMEM((CHUNK, D), jnp.float32),
        pltpu.SemaphoreType.DMA,
    ],
)
def _gather_kernel(idx_hbm, table_hbm, out_hbm, idx_v, rows_v, sem):
    wid = lax.axis_index("s") * 2 + lax.axis_index("c")
    base = wid * B_PER_W
    for c in range(NCHUNK):
        off = base + c * CHUNK
        pltpu.sync_copy(idx_hbm.at[pl.ds(off, CHUNK)], idx_v)
        pltpu.async_copy(table_hbm.at[idx_v], rows_v, sem).wait()
        pltpu.sync_copy(rows_v, out_hbm.at[pl.ds(off, CHUNK)])


def kernel(input, embeddings):
    idx_flat = input.reshape(-1).astype(jnp.int32)
    out = _gather_kernel(idx_flat, embeddings)
    return out.reshape(input.shape + (D,))


# speedup vs baseline: 2.4466x; 2.4466x over previous
import functools

import jax
import jax.numpy as jnp
from jax import lax
from jax.experimental import pallas as pl
from jax.experimental.pallas import tpu as pltpu
from jax.experimental.pallas import tpu_sc as plsc

D = 32
B = 4096 * 50
NW = 32
B_PER_W = B // NW
CHUNK = 1600
NCHUNK = B_PER_W // CHUNK
GROUPS = CHUNK // 16
ROWS = CHUNK // 4
OUT_ROWS = B * D // 128

_mesh = plsc.VectorSubcoreMesh(core_axis_name="c", subcore_axis_name="s")


@functools.partial(
    pl.kernel,
    mesh=_mesh,
    out_type=jax.ShapeDtypeStruct((OUT_ROWS, 128), jnp.float32),
    scratch_types=[
        pltpu.VMEM((B_PER_W,), jnp.int32),
        pltpu.VMEM((ROWS, 128), jnp.float32),
        pltpu.SemaphoreType.DMA,
    ],
)
def _gather_kernel(idx_hbm, table_hbm, out_hbm, idx_v, rows_v, sem):
    wid = lax.axis_index("s") * 2 + lax.axis_index("c")
    base = wid * B_PER_W
    out_base = base // 4
    pltpu.sync_copy(idx_hbm.at[pl.ds(base, B_PER_W)], idx_v)

    def chunk_body(c, _):
        def group_body(g, _):
            vec = idx_v[pl.ds((c * GROUPS + g) * 16, 16)]
            for j in range(16):
                r = vec[j]
                t = lax.shift_right_logical(r, 3)
                s = lax.bitwise_and(r, 7)
                k = g * 16 + j
                pltpu.async_copy(
                    table_hbm.at[t, s],
                    rows_v.at[k // 4, pl.ds((k % 4) * 32, 32)],
                    sem,
                )
            return 0

        lax.fori_loop(0, GROUPS, group_body, 0)
        pltpu.make_async_copy(
            out_hbm.at[pl.ds(0, ROWS)], rows_v, sem
        ).wait()
        o = pl.multiple_of(out_base + c * ROWS, 8)
        pltpu.sync_copy(rows_v, out_hbm.at[pl.ds(o, ROWS)])
        return 0

    lax.fori_loop(0, NCHUNK, chunk_body, 0)


def kernel(input, embeddings):
    idx_flat = input.reshape(-1).astype(jnp.int32)
    table3 = embeddings.reshape(125000, 8, 32)
    out = _gather_kernel(idx_flat, table3)
    return out.reshape(input.shape + (D,))
